# baseline (device time: 27842 ns/iter reference)
import jax
import jax.numpy as jnp
from jax import lax
from jax.experimental import pallas as pl
from jax.experimental.pallas import tpu as pltpu

N_DEV = 4
B, SQ, D = 2, 256, 512
H_LOCAL, DH = 4, 64
ROWS = B * SQ
Q4 = ROWS // N_DEV
EPS = 1e-5
F32 = jnp.float32
BF16 = jnp.bfloat16


def kernel(x, Wq, Wk, Wv, Wo, t_emb, W_mod, W_ff1, W_ff2):
    def body(
        x_hbm, wq_hbm, wk_hbm, wv_hbm, wo_hbm, temb_hbm, wmod_hbm,
        wff1_hbm, wff2_hbm, out_ref,
        xv_ref, wq_ref, wk_ref, wv_ref, wo_ref, temb_ref, wmod_ref,
        wff1_ref, wff2_ref,
        p1_ref,
        rs1_ref,
        x1ag_ref,
        p2_ref,
        rs2_ref,
        red2_ref,
        load_sems,
        send_sems, recv_sems,
    ):
        my = lax.axis_index("i")

        hbm = [x_hbm, temb_hbm, wmod_hbm, wq_hbm, wk_hbm, wv_hbm,
               wo_hbm, wff1_hbm, wff2_hbm]
        vmem = [xv_ref, temb_ref, wmod_ref, wq_ref, wk_ref, wv_ref,
                wo_ref, wff1_ref, wff2_ref]
        loads = []
        for i, (h, v) in enumerate(zip(hbm, vmem)):
            c = pltpu.make_async_copy(h, v, load_sems.at[i])
            c.start()
            loads.append(c)

        barrier_sem = pltpu.get_barrier_semaphore()
        for off in (1, 2, 3):
            pl.semaphore_signal(
                barrier_sem, inc=1,
                device_id=(lax.rem(my + off, N_DEV),),
                device_id_type=pl.DeviceIdType.MESH,
            )
        pl.semaphore_wait(barrier_sem, N_DEV - 1)

        def mk_rdma(phase, off, src, dst, dev, sem_idx=None):
            idx = phase * 3 + off - 1 if sem_idx is None else sem_idx
            return pltpu.make_async_remote_copy(
                src_ref=src,
                dst_ref=dst,
                send_sem=send_sems.at[idx],
                recv_sem=recv_sems.at[idx],
                device_id=(dev,),
                device_id_type=pl.DeviceIdType.MESH,
            )

        def exchange(phase, make_src, make_dst):
            rdmas = {}
            for off in (2, 1, 3):
                dev = lax.rem(my + off, N_DEV)
                rdma = mk_rdma(phase, off, make_src(off, dev),
                               make_dst(off, dev), dev)
                rdma.start()
                rdmas[off] = rdma
            return [rdmas[1], rdmas[2], rdmas[3]]

        def bsel(pair, qidx):
            return jnp.where(qidx >= 2, pair[1], pair[0])

        loads[0].wait()
        xb = [xv_ref[0], xv_ref[1]]
        ln1_stats = []
        for b in range(B):
            h = xb[b]
            m = jnp.mean(h, axis=-1, keepdims=True)
            v = jnp.mean((h - m) * (h - m), axis=-1, keepdims=True)
            ln1_stats.append((h - m) * lax.rsqrt(v + EPS))

        def my_quarter(pair):
            sel = jnp.where(my >= 2, pair[1], pair[0])
            return jnp.where(
                lax.rem(my, 2) == 0, sel[:Q4, :], sel[Q4:, :]
            )

        loads[1].wait()
        loads[2].wait()
        mod = []
        for b in range(B):
            mb = jnp.dot(
                temb_ref[b : b + 1, :], wmod_ref[...],
                preferred_element_type=F32,
            )
            mod.append([mb[:, i * D : (i + 1) * D] for i in range(6)])
        sa_, sha_, ga_, sm_, shm_, gm_ = (
            [mod[0][i], mod[1][i]] for i in range(6)
        )

        def ln_mod(h, scale, shift):
            m = jnp.mean(h, axis=-1, keepdims=True)
            v = jnp.mean((h - m) * (h - m), axis=-1, keepdims=True)
            return ((h - m) * lax.rsqrt(v + EPS)) * (1.0 + scale) + shift

        loads[3].wait()
        loads[4].wait()
        loads[5].wait()
        wq_b = wq_ref[...].astype(BF16)
        wk_b = wk_ref[...].astype(BF16)
        wv_b = wv_ref[...].astype(BF16)
        rs1_sends = []
        attn_parts = []
        for b in range(B):
            xm = (ln1_stats[b] * (1.0 + sa_[b]) + sha_[b]).astype(BF16)
            q = jnp.dot(xm, wq_b, preferred_element_type=F32).astype(BF16)
            k = jnp.dot(xm, wk_b, preferred_element_type=F32).astype(BF16)
            v = jnp.dot(xm, wv_b, preferred_element_type=F32).astype(BF16)
            outs = []
            for h in range(H_LOCAL):
                sl = slice(h * DH, (h + 1) * DH)
                s = lax.dot_general(
                    q[:, sl], k[:, sl], (((1,), (1,)), ((), ())),
                    preferred_element_type=F32,
                ) * 0.125
                mx = jnp.max(s, axis=-1, keepdims=True)
                p = jnp.exp(s - mx)
                l = jnp.sum(p, axis=-1, keepdims=True)
                outs.append(
                    jnp.dot(
                        p.astype(BF16), v[:, sl], preferred_element_type=F32
                    ) / l
                )
            o = jnp.concatenate(outs, axis=1).astype(BF16)
            if b == 0:
                loads[6].wait()
                wo_b = wo_ref[...].astype(BF16)
            part = jnp.dot(o, wo_b, preferred_element_type=F32)
            attn_parts.append(part)
            for qq in (2 * b, 2 * b + 1):
                blk = part[(qq - 2 * b) * Q4 : (qq - 2 * b + 1) * Q4, :]
                p1_ref[qq] = blk.astype(BF16)
                idx = jnp.maximum(lax.rem(qq - my + N_DEV, N_DEV) - 1, 0)
                r = mk_rdma(
                    0, 0, p1_ref.at[qq], rs1_ref.at[idx], qq, sem_idx=idx
                )
                @pl.when(my != qq)
                def _():
                    r.start()
                rs1_sends.append((r, qq))

        rs1_waits = [
            mk_rdma(0, off, p1_ref.at[0], rs1_ref.at[off - 1],
                    lax.rem(my + off, N_DEV))
            for off in (1, 3, 2)
        ]
        for r in rs1_waits:
            r.wait_recv()
        attn_my = my_quarter(attn_parts)
        for i in range(N_DEV - 1):
            attn_my = attn_my + rs1_ref[i].astype(F32)

        x1_my = my_quarter(xb) + bsel(ga_, my) * attn_my
        x1ag_ref[my] = x1_my.astype(BF16)
        ag1 = exchange(
            1,
            lambda off, dev: x1ag_ref.at[my],
            lambda off, dev: x1ag_ref.at[my],
        )

        loads[7].wait()
        loads[8].wait()
        wff1_b = wff1_ref[...].astype(BF16)
        wff2_b = wff2_ref[...].astype(BF16)

        def ffn_block(x1_blk, qidx):
            xm2 = ln_mod(x1_blk, bsel(sm_, qidx), bsel(shm_, qidx))
            hb = jnp.dot(xm2.astype(BF16), wff1_b, preferred_element_type=F32)
            hb = hb / (1.0 + jnp.exp(-hb))
            return jnp.dot(hb.astype(BF16), wff2_b, preferred_element_type=F32)

        ffn_my = ffn_block(x1_my, my)

        HD = D // 2
        halves = (slice(0, HD), slice(HD, D))
        rs2 = []
        for off in (1, 3, 2):
            ag1[off - 1].wait_recv()
            qidx = lax.rem(my - off + N_DEV, N_DEV)
            fblk = ffn_block(x1ag_ref[qidx].astype(F32), qidx)
            p2_ref[qidx] = fblk.astype(BF16)
            off_send = N_DEV - off
            pair = []
            for hf in (0, 1):
                r = mk_rdma(
                    0, 0,
                    p2_ref.at[qidx, :, pl.ds(hf * HD, HD)],
                    rs2_ref.at[off_send - 1, :, pl.ds(hf * HD, HD)],
                    qidx, sem_idx=12 + (off_send - 1) * 2 + hf,
                )
                r.start()
                pair.append(r)
            rs2.append(pair)

        ag2 = []
        out_half = []
        for hf in (0, 1):
            for r in rs2:
                r[hf].wait_recv()
            total2 = ffn_my[:, halves[hf]]
            for i in range(N_DEV - 1):
                total2 = total2 + rs2_ref[i, :, halves[hf]].astype(F32)
            oh = (
                x1_my[:, halves[hf]]
                + bsel(gm_, my)[:, halves[hf]] * total2
            )
            out_half.append(oh)
            red2_ref[my, :, halves[hf]] = oh.astype(BF16)
            for off in (1, 2, 3):
                dev = lax.rem(my + off, N_DEV)
                r = mk_rdma(
                    0, 0,
                    red2_ref.at[my, :, pl.ds(hf * HD, HD)],
                    red2_ref.at[my, :, pl.ds(hf * HD, HD)],
                    dev, sem_idx=18 + (off - 1) * 2 + hf,
                )
                r.start()
                ag2.append(r)
        out_ref[my // 2, pl.ds(lax.rem(my, 2) * Q4, Q4), :] = (
            jnp.concatenate(out_half, axis=1)
        )
        for off in (1, 3, 2):
            for hf in (0, 1):
                ag2[hf * 3 + (off - 1)].wait_recv()
            qidx = lax.rem(my - off + N_DEV, N_DEV)
            out_ref[qidx // 2, pl.ds(lax.rem(qidx, 2) * Q4, Q4), :] = (
                red2_ref[qidx].astype(F32)
            )

        for r, qq in rs1_sends:
            @pl.when(my != qq)
            def _():
                r.wait_send()
        for r in ag1:
            r.wait_send()
        for pair in rs2:
            for r in pair:
                r.wait_send()
        for r in ag2:
            r.wait_send()

    return pl.pallas_call(
        body,
        out_shape=jax.ShapeDtypeStruct((B, SQ, D), jnp.float32),
        in_specs=[pl.BlockSpec(memory_space=pltpu.MemorySpace.HBM)] * 9,
        out_specs=pl.BlockSpec(memory_space=pltpu.MemorySpace.VMEM),
        scratch_shapes=[
            pltpu.VMEM((B, SQ, D), F32),
            pltpu.VMEM((D, 256), F32),
            pltpu.VMEM((D, 256), F32),
            pltpu.VMEM((D, 256), F32),
            pltpu.VMEM((256, D), F32),
            pltpu.VMEM((B, 128), F32),
            pltpu.VMEM((128, 6 * D), F32),
            pltpu.VMEM((D, D), F32),
            pltpu.VMEM((D, D), F32),
            pltpu.VMEM((N_DEV, Q4, D), BF16),
            pltpu.VMEM((3, Q4, D), BF16),
            pltpu.VMEM((N_DEV, Q4, D), BF16),
            pltpu.VMEM((N_DEV, Q4, D), BF16),
            pltpu.VMEM((3, Q4, D), BF16),
            pltpu.VMEM((N_DEV, Q4, D), BF16),
            pltpu.SemaphoreType.DMA((9,)),
            pltpu.SemaphoreType.DMA((24,)),
            pltpu.SemaphoreType.DMA((24,)),
        ],
        compiler_params=pltpu.CompilerParams(collective_id=0),
    )(*(
        pltpu.with_memory_space_constraint(a, pltpu.MemorySpace.HBM)
        for a in (x, Wq, Wk, Wv, Wo, t_emb, W_mod, W_ff1, W_ff2)
    ))


# device time: 26214 ns/iter; 1.0621x vs baseline; 1.0621x over previous
import jax
import jax.numpy as jnp
from jax import lax
from jax.experimental import pallas as pl
from jax.experimental.pallas import tpu as pltpu

N_DEV = 4
B, SQ, D = 2, 256, 512
H_LOCAL, DH = 4, 64
ROWS = B * SQ
Q4 = ROWS // N_DEV
EPS = 1e-5
F32 = jnp.float32
BF16 = jnp.bfloat16


def kernel(x, Wq, Wk, Wv, Wo, t_emb, W_mod, W_ff1, W_ff2):
    def body(
        x_hbm, wq_hbm, wk_hbm, wv_hbm, wo_hbm, temb_hbm, wmod_hbm,
        wff1_hbm, wff2_hbm, out_ref,
        xv_ref, wq_ref, wk_ref, wv_ref, wo_ref, temb_ref, wmod_ref,
        wff1_ref, wff2_ref,
        p1_ref,
        rs1_ref,
        x1ag_ref,
        p2_ref,
        rs2_ref,
        red2_ref,
        load_sems,
        send_sems, recv_sems,
    ):
        my = lax.axis_index("i")

        hbm = [x_hbm, temb_hbm, wmod_hbm, wq_hbm, wk_hbm, wv_hbm,
               wo_hbm, wff1_hbm, wff2_hbm]
        vmem = [xv_ref, temb_ref, wmod_ref, wq_ref, wk_ref, wv_ref,
                wo_ref, wff1_ref, wff2_ref]
        loads = []
        for i, (h, v) in enumerate(zip(hbm, vmem)):
            c = pltpu.make_async_copy(h, v, load_sems.at[i])
            c.start()
            loads.append(c)

        barrier_sem = pltpu.get_barrier_semaphore()
        for off in (1, 2, 3):
            pl.semaphore_signal(
                barrier_sem, inc=1,
                device_id=(lax.rem(my + off, N_DEV),),
                device_id_type=pl.DeviceIdType.MESH,
            )
        pl.semaphore_wait(barrier_sem, N_DEV - 1)

        def mk_rdma(phase, off, src, dst, dev, sem_idx=None):
            idx = phase * 3 + off - 1 if sem_idx is None else sem_idx
            return pltpu.make_async_remote_copy(
                src_ref=src,
                dst_ref=dst,
                send_sem=send_sems.at[idx],
                recv_sem=recv_sems.at[idx],
                device_id=(dev,),
                device_id_type=pl.DeviceIdType.MESH,
            )

        def exchange(phase, make_src, make_dst):
            rdmas = []
            for off in (1, 2, 3):
                dev = lax.rem(my + off, N_DEV)
                rdma = mk_rdma(phase, off, make_src(off, dev),
                               make_dst(off, dev), dev)
                rdma.start()
                rdmas.append(rdma)
            return rdmas

        def bsel(pair, qidx):
            return jnp.where(qidx >= 2, pair[1], pair[0])

        loads[0].wait()
        xb = [xv_ref[0], xv_ref[1]]
        ln1_stats = []
        for b in range(B):
            h = xb[b]
            m = jnp.mean(h, axis=-1, keepdims=True)
            v = jnp.mean((h - m) * (h - m), axis=-1, keepdims=True)
            ln1_stats.append((h - m) * lax.rsqrt(v + EPS))

        def my_quarter(pair):
            sel = jnp.where(my >= 2, pair[1], pair[0])
            return jnp.where(
                lax.rem(my, 2) == 0, sel[:Q4, :], sel[Q4:, :]
            )

        loads[1].wait()
        loads[2].wait()
        mod = []
        for b in range(B):
            mb = jnp.dot(
                temb_ref[b : b + 1, :], wmod_ref[...],
                preferred_element_type=F32,
            )
            mod.append([mb[:, i * D : (i + 1) * D] for i in range(6)])
        sa_, sha_, ga_, sm_, shm_, gm_ = (
            [mod[0][i], mod[1][i]] for i in range(6)
        )

        def ln_mod(h, scale, shift):
            m = jnp.mean(h, axis=-1, keepdims=True)
            v = jnp.mean((h - m) * (h - m), axis=-1, keepdims=True)
            return ((h - m) * lax.rsqrt(v + EPS)) * (1.0 + scale) + shift

        loads[3].wait()
        loads[4].wait()
        loads[5].wait()
        wq_b = wq_ref[...].astype(BF16)
        wk_b = wk_ref[...].astype(BF16)
        wv_b = wv_ref[...].astype(BF16)
        rs1_sends = []
        attn_parts = []
        for b in range(B):
            xm = (ln1_stats[b] * (1.0 + sa_[b]) + sha_[b]).astype(BF16)
            q = jnp.dot(xm, wq_b, preferred_element_type=F32).astype(BF16)
            k = jnp.dot(xm, wk_b, preferred_element_type=F32).astype(BF16)
            v = jnp.dot(xm, wv_b, preferred_element_type=F32).astype(BF16)
            outs = []
            for h in range(H_LOCAL):
                sl = slice(h * DH, (h + 1) * DH)
                s = lax.dot_general(
                    q[:, sl], k[:, sl], (((1,), (1,)), ((), ())),
                    preferred_element_type=F32,
                ) * 0.125
                mx = jnp.max(s, axis=-1, keepdims=True)
                p = jnp.exp(s - mx)
                l = jnp.sum(p, axis=-1, keepdims=True)
                outs.append(
                    jnp.dot(
                        p.astype(BF16), v[:, sl], preferred_element_type=F32
                    ) / l
                )
            o = jnp.concatenate(outs, axis=1).astype(BF16)
            if b == 0:
                loads[6].wait()
                wo_b = wo_ref[...].astype(BF16)
            part = jnp.dot(o, wo_b, preferred_element_type=F32)
            attn_parts.append(part)
            for qq in (2 * b, 2 * b + 1):
                blk = part[(qq - 2 * b) * Q4 : (qq - 2 * b + 1) * Q4, :]
                p1_ref[qq] = blk.astype(BF16)
                idx = jnp.maximum(lax.rem(qq - my + N_DEV, N_DEV) - 1, 0)
                r = mk_rdma(
                    0, 0, p1_ref.at[qq], rs1_ref.at[idx], qq, sem_idx=idx
                )
                @pl.when(my != qq)
                def _():
                    r.start()
                rs1_sends.append((r, qq))

        rs1_waits = [
            mk_rdma(0, off, p1_ref.at[0], rs1_ref.at[off - 1],
                    lax.rem(my + off, N_DEV))
            for off in (1, 3, 2)
        ]
        for r in rs1_waits:
            r.wait_recv()
        attn_my = my_quarter(attn_parts)
        for i in range(N_DEV - 1):
            attn_my = attn_my + rs1_ref[i].astype(F32)

        x1_my = my_quarter(xb) + bsel(ga_, my) * attn_my
        x1ag_ref[my] = x1_my.astype(BF16)
        ag1 = exchange(
            1,
            lambda off, dev: x1ag_ref.at[my],
            lambda off, dev: x1ag_ref.at[my],
        )

        loads[7].wait()
        loads[8].wait()
        wff1_b = wff1_ref[...].astype(BF16)
        wff2_b = wff2_ref[...].astype(BF16)

        def ffn_block(x1_blk, qidx):
            xm2 = ln_mod(x1_blk, bsel(sm_, qidx), bsel(shm_, qidx))
            hb = jnp.dot(xm2.astype(BF16), wff1_b, preferred_element_type=F32)
            hb = hb / (1.0 + jnp.exp(-hb))
            return jnp.dot(hb.astype(BF16), wff2_b, preferred_element_type=F32)

        ffn_my = ffn_block(x1_my, my)

        HD = D // 2
        halves = (slice(0, HD), slice(HD, D))
        rs2 = []
        for off in (1, 3, 2):
            ag1[off - 1].wait_recv()
            qidx = lax.rem(my - off + N_DEV, N_DEV)
            fblk = ffn_block(x1ag_ref[qidx].astype(F32), qidx)
            p2_ref[qidx] = fblk.astype(BF16)
            off_send = N_DEV - off
            pair = []
            for hf in (0, 1):
                r = mk_rdma(
                    0, 0,
                    p2_ref.at[qidx, :, pl.ds(hf * HD, HD)],
                    rs2_ref.at[off_send - 1, :, pl.ds(hf * HD, HD)],
                    qidx, sem_idx=12 + (off_send - 1) * 2 + hf,
                )
                r.start()
                pair.append(r)
            rs2.append(pair)

        ag2 = []
        out_half = []
        for hf in (0, 1):
            for r in rs2:
                r[hf].wait_recv()
            total2 = ffn_my[:, halves[hf]]
            for i in range(N_DEV - 1):
                total2 = total2 + rs2_ref[i, :, halves[hf]].astype(F32)
            oh = (
                x1_my[:, halves[hf]]
                + bsel(gm_, my)[:, halves[hf]] * total2
            )
            out_half.append(oh)
            red2_ref[my, :, halves[hf]] = oh.astype(BF16)
            for off in (1, 2, 3):
                dev = lax.rem(my + off, N_DEV)
                r = mk_rdma(
                    0, 0,
                    red2_ref.at[my, :, pl.ds(hf * HD, HD)],
                    red2_ref.at[my, :, pl.ds(hf * HD, HD)],
                    dev, sem_idx=18 + (off - 1) * 2 + hf,
                )
                r.start()
                ag2.append(r)
        out_ref[my // 2, pl.ds(lax.rem(my, 2) * Q4, Q4), :] = (
            jnp.concatenate(out_half, axis=1)
        )
        for off in (1, 3, 2):
            for hf in (0, 1):
                ag2[hf * 3 + (off - 1)].wait_recv()
            qidx = lax.rem(my - off + N_DEV, N_DEV)
            out_ref[qidx // 2, pl.ds(lax.rem(qidx, 2) * Q4, Q4), :] = (
                red2_ref[qidx].astype(F32)
            )

        for r, qq in rs1_sends:
            @pl.when(my != qq)
            def _():
                r.wait_send()
        for r in ag1:
            r.wait_send()
        for pair in rs2:
            for r in pair:
                r.wait_send()
        for r in ag2:
            r.wait_send()

    return pl.pallas_call(
        body,
        out_shape=jax.ShapeDtypeStruct((B, SQ, D), jnp.float32),
        in_specs=[pl.BlockSpec(memory_space=pltpu.MemorySpace.HBM)] * 9,
        out_specs=pl.BlockSpec(memory_space=pltpu.MemorySpace.VMEM),
        scratch_shapes=[
            pltpu.VMEM((B, SQ, D), F32),
            pltpu.VMEM((D, 256), F32),
            pltpu.VMEM((D, 256), F32),
            pltpu.VMEM((D, 256), F32),
            pltpu.VMEM((256, D), F32),
            pltpu.VMEM((B, 128), F32),
            pltpu.VMEM((128, 6 * D), F32),
            pltpu.VMEM((D, D), F32),
            pltpu.VMEM((D, D), F32),
            pltpu.VMEM((N_DEV, Q4, D), BF16),
            pltpu.VMEM((3, Q4, D), BF16),
            pltpu.VMEM((N_DEV, Q4, D), BF16),
            pltpu.VMEM((N_DEV, Q4, D), BF16),
            pltpu.VMEM((3, Q4, D), BF16),
            pltpu.VMEM((N_DEV, Q4, D), BF16),
            pltpu.SemaphoreType.DMA((9,)),
            pltpu.SemaphoreType.DMA((24,)),
            pltpu.SemaphoreType.DMA((24,)),
        ],
        compiler_params=pltpu.CompilerParams(collective_id=0),
    )(*(
        pltpu.with_memory_space_constraint(a, pltpu.MemorySpace.HBM)
        for a in (x, Wq, Wk, Wv, Wo, t_emb, W_mod, W_ff1, W_ff2)
    ))
